# baseline (speedup 1.0000x reference)
import jax
import jax.numpy as jnp
from jax.experimental import pallas as pl
from jax.experimental.pallas import tpu as pltpu

N_ = 2048
E_ = 8192
T_ = 4096
F_ = 16

_f32 = jnp.float32
_bf16 = jnp.bfloat16


def _params():
    return pltpu.CompilerParams(dimension_semantics=("arbitrary",))


def _dg(a, b, dims):
    return jax.lax.dot_general(a, b, (dims, ((), ())),
                               preferred_element_type=_f32)


_UJ = 16
_UCB = T_ // _UJ


def _ut_kernel(b2_ref, hb_ref, o_ref):
    o_ref[...] = _dg(hb_ref[...].astype(_f32), b2_ref[...],
                     ((0,), (0,))).astype(_bf16)


def _pallas_ut(b2, hb):
    return pl.pallas_call(
        _ut_kernel,
        grid=(_UJ,),
        in_specs=[
            pl.BlockSpec((E_, _UCB), lambda j: (0, j)),
            pl.BlockSpec((E_, F_), lambda j: (0, 0)),
        ],
        out_specs=pl.BlockSpec((F_, _UCB), lambda j: (0, j)),
        out_shape=jax.ShapeDtypeStruct((F_, T_), _bf16),
        compiler_params=_params(),
    )(b2, hb)


_ZJ = 8
_ZEB = E_ // _ZJ


def _y2t_kernel(ut_ref, b2_ref, o_ref):
    o_ref[...] = _dg(ut_ref[...].astype(_f32), b2_ref[...],
                     ((1,), (1,))).astype(_bf16)


def _pallas_y2t(ut, b2):
    return pl.pallas_call(
        _y2t_kernel,
        grid=(_ZJ,),
        in_specs=[
            pl.BlockSpec((F_, T_), lambda j: (0, 0)),
            pl.BlockSpec((_ZEB, T_), lambda j: (j, 0)),
        ],
        out_specs=pl.BlockSpec((F_, _ZEB), lambda j: (0, j)),
        out_shape=jax.ShapeDtypeStruct((F_, E_), _bf16),
        compiler_params=_params(),
    )(ut, b2)


def kernel(x, incidence_1, incidence_2, w0_0, w0_1, w0_2, w1_0, w1_1, w1_2,
           w2_0, w2_1, w2_2, w3_0, w3_1, w3_2):
    b1, b2 = incidence_1, incidence_2
    ws = [(w0_0, w0_1, w0_2), (w1_0, w1_1, w1_2),
          (w2_0, w2_1, w2_2), (w3_0, w3_1, w3_2)]
    h = x
    for li in range(4):
        w0, w1, w2 = ws[li]
        if li == 1:
            hb = h.astype(_bf16)
            ut = _pallas_ut(b2, hb)
            y2t = _pallas_y2t(ut, b2)
            y2 = jax.lax.dot_general(y2t.astype(_f32), w2,
                                     (((0,), (0,)), ((), ())))
            y0 = b1.T @ (b1 @ h) @ w0
            y1 = h @ w1
            h = jnp.tanh(y0 + y1 + y2)
        else:
            y0 = b1.T @ (b1 @ h) @ w0
            y1 = h @ w1
            y2 = b2 @ (b2.T @ h) @ w2
            h = jnp.tanh(y0 + y1 + y2)
    return h
